# Initial kernel scaffold; baseline (speedup 1.0000x reference)
import jax
import jax.numpy as jnp
from jax.experimental import pallas as pl


def kernel(f0, pitch_embed, uv_embed, W, b):
    raise NotImplementedError("write your pallas kernel here")


# speedup vs baseline: 15.2370x; 15.2370x over previous
import functools

import jax
import jax.numpy as jnp
import numpy as np
from jax.experimental import pallas as pl
from jax.experimental.pallas import tpu as pltpu

N_BINS = 256
F0_MIN = 50.0
F0_MAX = 1100.0
OUT = 256

_MEL_MIN = 1127.0 * np.log(1.0 + F0_MIN / 700.0)
_MEL_MAX = 1127.0 * np.log(1.0 + F0_MAX / 700.0)
_MEL_SCALE = (N_BINS - 1) / (_MEL_MAX - _MEL_MIN)


def _body(f0_ref, pe_ref, uv_ref, w_ref, b_ref, out_ref):
    f0 = f0_ref[0]
    af0 = jnp.abs(f0)
    mel = 1127.0 * jnp.log1p(af0 / 700.0)
    binsf = (mel - _MEL_MIN) * _MEL_SCALE
    bins = jnp.clip(binsf.astype(jnp.int32), 0, N_BINS - 1)
    uvf = (af0 > 10.0).astype(jnp.float32)
    flog = jnp.log1p(af0)

    t = f0.shape[-1]
    bins_t = bins.reshape(t, 1)
    uvf_t = uvf.reshape(t, 1)
    flog_t = flog.reshape(t, 1)

    iota = jax.lax.broadcasted_iota(jnp.int32, (t, N_BINS), 1)
    onehot = (iota == bins_t).astype(jnp.bfloat16)
    pitch = jnp.dot(onehot, pe_ref[...].astype(jnp.bfloat16),
                    preferred_element_type=jnp.float32)

    uv0 = uv_ref[0:1]
    uvd = uv_ref[1:2] - uv_ref[0:1]
    res = pitch + (uv0 + b_ref[...]) + uvf_t * uvd + flog_t * w_ref[...]
    out_ref[0] = res


def kernel(f0, pitch_embed, uv_embed, W, b):
    B, T = f0.shape
    n_elems = B * T
    blk = 4096
    grid = n_elems // blk
    f0_r = f0.reshape(grid, 1, blk)
    w_row = W.reshape(1, OUT)
    b_row = b.reshape(1, OUT)

    out = pl.pallas_call(
        _body,
        grid=(grid,),
        in_specs=[
            pl.BlockSpec((1, 1, blk), lambda i: (i, 0, 0)),
            pl.BlockSpec((N_BINS, OUT), lambda i: (0, 0)),
            pl.BlockSpec((2, OUT), lambda i: (0, 0)),
            pl.BlockSpec((1, OUT), lambda i: (0, 0)),
            pl.BlockSpec((1, OUT), lambda i: (0, 0)),
        ],
        out_specs=pl.BlockSpec((1, blk, OUT), lambda i: (i, 0, 0)),
        out_shape=jax.ShapeDtypeStruct((grid, blk, OUT), jnp.float32),
    )(f0_r, pitch_embed, uv_embed, w_row, b_row)
    return out.reshape(B, T, OUT)
